# Initial kernel scaffold; baseline (speedup 1.0000x reference)
import jax
import jax.numpy as jnp
from jax.experimental import pallas as pl


def kernel(primals_1, primals_2, primals_3, primals_4):
    raise NotImplementedError("write your pallas kernel here")


# speedup vs baseline: 1.0634x; 1.0634x over previous
import functools

import jax
import jax.numpy as jnp
from jax import lax
from jax.experimental import pallas as pl
from jax.experimental.pallas import tpu as pltpu
from jax.experimental.pallas import tpu_sc as plsc

jax.config.update("jax_enable_x64", True)

_NC = 2
_NS = 16
_NW = _NC * _NS
_LANES = 16
_ROWS = 11 * 64
_RPW = 24
_PADROWS = _NW * _RPW

_mesh = plsc.VectorSubcoreMesh(core_axis_name="c", subcore_axis_name="s")


@functools.partial(
    pl.kernel,
    mesh=_mesh,
    out_type=[
        jax.ShapeDtypeStruct((_PADROWS, 128), jnp.int32),
        jax.ShapeDtypeStruct((16, 128), jnp.float32),
    ],
    scratch_types=[
        pltpu.VMEM((_RPW, _LANES), jnp.int32),
        pltpu.VMEM((_RPW, 128), jnp.int32),
        pltpu.VMEM((16,), jnp.int32),
        pltpu.VMEM((16, 128), jnp.float32),
        pltpu.SemaphoreType.DMA,
    ],
)
def _sc_mask_gather(thr_hbm, idx_hbm, table_hbm, mask_out, rows_out,
                    thr_v, mask_v, idx_v, rows_v, sem):
    wid = lax.axis_index("s") * _NC + lax.axis_index("c")
    base = wid * _RPW
    pltpu.sync_copy(thr_hbm.at[pl.ds(base, _RPW)], thr_v)
    col0 = lax.iota(jnp.int32, _LANES)

    def body(r, carry):
        tvec = thr_v[r]
        for k in range(128 // _LANES):
            col = col0 + (k * _LANES)
            val = jnp.where(col < tvec, jnp.int32(1), jnp.int32(0))
            mask_v[r, pl.ds(k * _LANES, _LANES)] = val
        return carry

    lax.fori_loop(0, _RPW, body, 0)
    pltpu.sync_copy(mask_v, mask_out.at[pl.ds(base, _RPW)])

    @pl.when(wid == 0)
    def _gather():
        pltpu.sync_copy(idx_hbm, idx_v)
        pltpu.async_copy(table_hbm.at[idx_v], rows_v, sem).wait()
        pltpu.sync_copy(rows_v, rows_out)


def kernel(primals_1, primals_2, primals_3, primals_4):
    p2 = primals_2.astype(jnp.int32)
    ct1 = primals_3.astype(jnp.int32)
    thr1d = jnp.pad(p2[:, :, 0].reshape(-1), (0, _PADROWS - _ROWS))
    thr = jnp.broadcast_to(thr1d[:, None], (_PADROWS, _LANES))
    select_2 = p2[:, 0, 2]
    idx16 = jnp.pad(select_2, (0, 16 - select_2.shape[0]))
    mask_i32, rows = _sc_mask_gather(thr, idx16, primals_4)
    lt = mask_i32[:_ROWS, :120].astype(jnp.bool_).reshape(11, 64, 120)
    index = rows[:11]
    z0 = jnp.zeros((11, 6, 128), jnp.float64)
    z1 = jnp.zeros((11, 32, 128), jnp.float64)
    z2 = jnp.zeros((11, 128), jnp.float64)
    return (primals_1, ct1, z0, z1, z2, lt, index, select_2)
